# Initial kernel scaffold; baseline (speedup 1.0000x reference)
import jax
import jax.numpy as jnp
from jax.experimental import pallas as pl


def kernel(coords, feats, W_in, b_in, W_lvl, b_lvl, W_sem, queries, W_cls):
    raise NotImplementedError("write your pallas kernel here")


# speedup vs baseline: 1.8536x; 1.8536x over previous
import jax
import jax.numpy as jnp
from jax.experimental import pallas as pl
from jax.experimental.pallas import tpu as pltpu

_B, _N, _CIN, _D, _Q, _NCLS = 4, 30000, 4, 32, 100, 20
_N2 = _N // 16
_BLK = 3000
_NB = _N // _BLK

_INV_SQRT_D = 1.0 / float(_D) ** 0.5


def _fused(c_ref, f_ref, c2_ref, f2_ref, Wc_ref, Wf_ref, bin_ref,
           W0_ref, b0_ref, W2_ref, b2_ref, Wsem_ref, q_ref, Wcls_ref,
           logits_ref, masks_ref, sem_ref, qf_scr):
    j = pl.program_id(1)

    @pl.when(j == 0)
    def _coarse_path():
        x2 = jnp.maximum(
            c2_ref[0] @ Wc_ref[...] + f2_ref[0] @ Wf_ref[...] + bin_ref[...],
            0.0)
        l2 = jnp.maximum(x2 @ W2_ref[...] + b2_ref[...], 0.0)
        sem_ref[0] = l2 @ Wsem_ref[...]
        scores = jax.lax.dot_general(
            q_ref[...], l2, (((1,), (1,)), ((), ()))) * _INV_SQRT_D
        scores = scores - jnp.max(scores, axis=-1, keepdims=True)
        e = jnp.exp(scores)
        attn = e / jnp.sum(e, axis=-1, keepdims=True)
        qf = attn @ l2
        qf_scr[...] = qf
        logits_ref[0] = qf @ Wcls_ref[...]

    x = jnp.maximum(
        c_ref[0] @ Wc_ref[...] + f_ref[0] @ Wf_ref[...] + bin_ref[...], 0.0)
    l0 = jnp.maximum(x @ W0_ref[...] + b0_ref[...], 0.0)
    masks_ref[0] = jax.lax.dot_general(
        l0, qf_scr[...], (((1,), (1,)), ((), ())))


def kernel(coords, feats, W_in, b_in, W_lvl, b_lvl, W_sem, queries, W_cls):
    coords2 = coords[:, ::16]
    feats2 = feats[:, ::16]
    Wc, Wf = W_in[:3], W_in[3:]
    b_in2 = b_in.reshape(1, _D)
    W0, W2 = W_lvl[0], W_lvl[2]
    b0, b2 = b_lvl[0].reshape(1, _D), b_lvl[2].reshape(1, _D)

    full = lambda *shape: pl.BlockSpec(shape, lambda b, j: (0,) * len(shape))
    per_b = lambda *shape: pl.BlockSpec(shape, lambda b, j: (b, 0, 0))

    logits, masks, sem = pl.pallas_call(
        _fused,
        grid=(_B, _NB),
        in_specs=[
            pl.BlockSpec((1, _BLK, 3), lambda b, j: (b, j, 0)),
            pl.BlockSpec((1, _BLK, _CIN), lambda b, j: (b, j, 0)),
            per_b(1, _N2, 3),
            per_b(1, _N2, _CIN),
            full(3, _D),
            full(_CIN, _D),
            full(1, _D),
            full(_D, _D),
            full(1, _D),
            full(_D, _D),
            full(1, _D),
            full(_D, _NCLS),
            full(_Q, _D),
            full(_D, _NCLS + 1),
        ],
        out_specs=[
            per_b(1, _Q, _NCLS + 1),
            pl.BlockSpec((1, _BLK, _Q), lambda b, j: (b, j, 0)),
            per_b(1, _N2, _NCLS),
        ],
        out_shape=[
            jax.ShapeDtypeStruct((_B, _Q, _NCLS + 1), jnp.float32),
            jax.ShapeDtypeStruct((_B, _N, _Q), jnp.float32),
            jax.ShapeDtypeStruct((_B, _N2, _NCLS), jnp.float32),
        ],
        scratch_shapes=[pltpu.VMEM((_Q, _D), jnp.float32)],
    )(coords, feats, coords2, feats2, Wc, Wf, b_in2,
      W0, b0, W2, b2, W_sem, queries, W_cls)
    return (logits, masks, sem)
